# baseline (speedup 1.0000x reference)
import jax
import jax.numpy as jnp
from jax import lax
from jax.experimental import pallas as pl
from jax.experimental.pallas import tpu as pltpu
from jax.experimental.pallas import tpu_sc as plsc

_B = 16384
_D = 64
_W = 128
_NC = 2
_NS = 16
_NW = _NC * _NS
_BPW = _B // _NW
_CHUNK = 128
_NCHUNK = _BPW // _CHUNK
_PHASE = 256
_NPHASE = _BPW // _PHASE
_CPP = _PHASE // _CHUNK
_LANES = 16


def _bpr_body(user_h, item_i_h, item_j_h, uw_h, iw_h, out_i_h, out_j_h,
              idx_u, idx_i, idx_j, pair_u, pair_i, pair_j,
              rows_u, rows_i, rows_j, out_i_v, out_j_v, sem):
    c = lax.axis_index("c")
    s = lax.axis_index("s")
    wid = s * _NC + c
    base = wid * _BPW
    chunk_base = wid * _NCHUNK

    pltpu.sync_copy(user_h.at[pl.ds(chunk_base, _NCHUNK)], idx_u)
    pltpu.sync_copy(item_i_h.at[pl.ds(chunk_base, _NCHUNK)], idx_i)
    pltpu.sync_copy(item_j_h.at[pl.ds(chunk_base, _NCHUNK)], idx_j)

    for ch in range(_NCHUNK):
        for v in range(_CHUNK // _LANES):
            sl = pl.ds(v * _LANES, _LANES)
            pair_u[ch, sl] = idx_u[ch, sl] >> 1
            pair_i[ch, sl] = idx_i[ch, sl] >> 1
            pair_j[ch, sl] = idx_j[ch, sl] >> 1

    lanes = lax.iota(jnp.int32, _LANES)

    for p in range(_NPHASE):
        copies = []
        for ch in range(_CPP):
            gch = p * _CPP + ch
            dst = pl.ds(ch * _CHUNK, _CHUNK)
            copies.append(pltpu.async_copy(
                uw_h.at[pair_u.at[gch]], rows_u.at[dst], sem))
            copies.append(pltpu.async_copy(
                iw_h.at[pair_i.at[gch]], rows_i.at[dst], sem))
            copies.append(pltpu.async_copy(
                iw_h.at[pair_j.at[gch]], rows_j.at[dst], sem))
        for cp in copies:
            cp.wait()

        for g in range(_PHASE // _LANES):
            gch = p * _CPP + g * _LANES // _CHUNK
            isl = pl.ds((g * _LANES) % _CHUNK, _LANES)
            rows16 = g * _LANES + lanes
            half_u = (idx_u[gch, isl] & 1) * _D
            half_i = (idx_i[gch, isl] & 1) * _D
            half_j = (idx_j[gch, isl] & 1) * _D
            zero = jnp.zeros((_LANES,), jnp.float32)

            def d_body(d, carry):
                acc_i, acc_j, cu, ci, cj = carry
                u = plsc.load_gather(rows_u, [rows16, cu])
                vi = plsc.load_gather(rows_i, [rows16, ci])
                vj = plsc.load_gather(rows_j, [rows16, cj])
                return (acc_i + u * vi, acc_j + u * vj,
                        cu + 1, ci + 1, cj + 1)

            acc_i, acc_j, _, _, _ = lax.fori_loop(
                0, _D, d_body, (zero, zero, half_u, half_i, half_j),
                unroll=8)
            out = pl.ds(p * _PHASE + g * _LANES, _LANES)
            out_i_v[out] = acc_i
            out_j_v[out] = acc_j

    pltpu.sync_copy(out_i_v, out_i_h.at[pl.ds(base, _BPW)])
    pltpu.sync_copy(out_j_v, out_j_h.at[pl.ds(base, _BPW)])


_bpr_call = pl.kernel(
    _bpr_body,
    out_type=(
        jax.ShapeDtypeStruct((_B,), jnp.float32),
        jax.ShapeDtypeStruct((_B,), jnp.float32),
    ),
    mesh=plsc.VectorSubcoreMesh(
        core_axis_name="c", subcore_axis_name="s",
        num_cores=_NC, num_subcores=_NS,
    ),
    compiler_params=pltpu.CompilerParams(
        needs_layout_passes=False, use_tc_tiling_on_sc=True),
    scratch_types=[
        pltpu.VMEM((_NCHUNK, _CHUNK), jnp.int32),
        pltpu.VMEM((_NCHUNK, _CHUNK), jnp.int32),
        pltpu.VMEM((_NCHUNK, _CHUNK), jnp.int32),
        pltpu.VMEM((_NCHUNK, _CHUNK), jnp.int32),
        pltpu.VMEM((_NCHUNK, _CHUNK), jnp.int32),
        pltpu.VMEM((_NCHUNK, _CHUNK), jnp.int32),
        pltpu.VMEM((_PHASE, _W), jnp.float32),
        pltpu.VMEM((_PHASE, _W), jnp.float32),
        pltpu.VMEM((_PHASE, _W), jnp.float32),
        pltpu.VMEM((_BPW,), jnp.float32),
        pltpu.VMEM((_BPW,), jnp.float32),
        pltpu.SemaphoreType.DMA,
    ],
)


def kernel(user, item_i, item_j, embed_user_weight, embed_item_weight):
    shape2d = (_B // _CHUNK, _CHUNK)
    u2 = user.astype(jnp.int32).reshape(shape2d)
    i2 = item_i.astype(jnp.int32).reshape(shape2d)
    j2 = item_j.astype(jnp.int32).reshape(shape2d)
    uw2 = embed_user_weight.reshape(-1, _W)
    iw2 = embed_item_weight.reshape(-1, _W)
    return _bpr_call(u2, i2, j2, uw2, iw2)


# speedup vs baseline: 2.1392x; 2.1392x over previous
import jax
import jax.numpy as jnp
from jax import lax
from jax.experimental import pallas as pl
from jax.experimental.pallas import tpu as pltpu
from jax.experimental.pallas import tpu_sc as plsc

_B = 16384
_D = 64
_W = 128
_NC = 2
_NS = 16
_NW = _NC * _NS
_LANES = 16
_NBUF = 4

_UPT = _B // _NW
_CPT = 2 * _B // _NW
_CHUNK = 128


def _wid():
    return lax.axis_index("s") * _NC + lax.axis_index("c")


def _stream_body(table_h, rl_v_h, slot_v_h, fetch_h, nf_h, stag_h,
                 rl_v, slot_v, fetch_v, nf_v, rows_out, ring, sem, osem,
                 *, n_per_tile):
    nchunk = n_per_tile // _CHUNK
    w = _wid()
    pltpu.sync_copy(rl_v_h.at[pl.ds(w * nchunk, nchunk)], rl_v)
    pltpu.sync_copy(slot_v_h.at[pl.ds(w * nchunk, nchunk)], slot_v)
    pltpu.sync_copy(fetch_h.at[pl.ds(w * nchunk, nchunk)], fetch_v)
    pltpu.sync_copy(nf_h.at[pl.ds(w, 1)], nf_v)

    lanes = lax.iota(jnp.int32, _LANES)

    def bcast(x):
        return jnp.full((_LANES,), x, jnp.int32)

    def fire(s):
        c = plsc.load_gather(
            fetch_v, [bcast(s >> 7), bcast(s & (_CHUNK - 1))])[0]
        pltpu.async_copy(
            table_h.at[:, pl.ds(c * _CHUNK, _CHUNK)],
            ring.at[s % _NBUF], sem)

    nf = nf_v[0, pl.ds(0, _LANES)][0]
    for s0 in range(_NBUF - 1):
        @pl.when(s0 < nf)
        def _():
            fire(jnp.int32(s0))

    def group_body(g, cur):
        ch = g >> 3
        off = (g & 7) * _LANES + lanes
        slot16 = plsc.load_gather(slot_v, [bcast(ch), off])
        rl16 = plsc.load_gather(rl_v, [bcast(ch), off])
        for k in range(_LANES):
            slot = slot16[k]

            def advance(c):
                pltpu.make_async_copy(
                    table_h.at[:, pl.ds(0, _CHUNK)], ring.at[0], sem
                ).wait()
                @pl.when(c + _NBUF - 1 < nf)
                def _():
                    fire(c + _NBUF - 1)
                return c + 1

            cur = lax.while_loop(lambda c: c <= slot, advance, cur)
            ringsel = bcast(slot % _NBUF)
            col = bcast(rl16[k])
            row16 = bcast(g * 8 + (k >> 1))
            for qq in range(_D // _LANES):
                vals = plsc.load_gather(
                    ring, [ringsel, qq * _LANES + lanes, col])
                plsc.store_scatter(
                    rows_out,
                    [row16, (k & 1) * _D + qq * _LANES + lanes], vals)
        return cur

    lax.fori_loop(0, n_per_tile // _LANES, group_body, jnp.int32(0))

    out_rows = n_per_tile // 2
    pltpu.async_copy(
        rows_out, stag_h.at[pl.ds(w * out_rows, out_rows)], osem).wait()


def _make_stream(n_per_tile, n_total):
    nchunk = n_per_tile // _CHUNK
    import functools
    body = functools.partial(_stream_body, n_per_tile=n_per_tile)
    return pl.kernel(
        body,
        out_type=jax.ShapeDtypeStruct((n_total // 2, _W), jnp.float32),
        mesh=plsc.VectorSubcoreMesh(
            core_axis_name="c", subcore_axis_name="s",
            num_cores=_NC, num_subcores=_NS,
        ),
        compiler_params=pltpu.CompilerParams(
            needs_layout_passes=False, use_tc_tiling_on_sc=True),
        scratch_types=[
            pltpu.VMEM((nchunk, _CHUNK), jnp.int32),
            pltpu.VMEM((nchunk, _CHUNK), jnp.int32),
            pltpu.VMEM((nchunk, _CHUNK), jnp.int32),
            pltpu.VMEM((1, _CHUNK), jnp.int32),
            pltpu.VMEM((n_per_tile // 2, _W), jnp.float32),
            pltpu.VMEM((_NBUF, _D, _CHUNK), jnp.float32),
            pltpu.SemaphoreType.DMA,
            pltpu.SemaphoreType.DMA,
        ],
    )


_stream_user = _make_stream(_UPT, _B)
_stream_item = _make_stream(_CPT, 2 * _B)


def _dot_body(inv_u_h, inv_i_h, inv_j_h, su_h, si_h, sj_h, out_i_h, out_j_h,
              inv_u, inv_i, inv_j, pair_u, pair_i, pair_j,
              rows_u, rows_i, rows_j, out_i_v, out_j_v, sem):
    w = _wid()
    bpw = _B // _NW
    nchunk = bpw // _CHUNK
    base = w * bpw

    pltpu.sync_copy(inv_u_h.at[pl.ds(w * nchunk, nchunk)], inv_u)
    pltpu.sync_copy(inv_i_h.at[pl.ds(w * nchunk, nchunk)], inv_i)
    pltpu.sync_copy(inv_j_h.at[pl.ds(w * nchunk, nchunk)], inv_j)

    for ch in range(nchunk):
        for v in range(_CHUNK // _LANES):
            sl = pl.ds(v * _LANES, _LANES)
            pair_u[ch, sl] = inv_u[ch, sl] >> 1
            pair_i[ch, sl] = inv_i[ch, sl] >> 1
            pair_j[ch, sl] = inv_j[ch, sl] >> 1

    lanes = lax.iota(jnp.int32, _LANES)
    nphase = 2
    ppb = bpw // nphase
    cpp = ppb // _CHUNK

    for p in range(nphase):
        copies = []
        for ch in range(cpp):
            gch = p * cpp + ch
            dst = pl.ds(ch * _CHUNK, _CHUNK)
            copies.append(pltpu.async_copy(
                su_h.at[pair_u.at[gch]], rows_u.at[dst], sem))
            copies.append(pltpu.async_copy(
                si_h.at[pair_i.at[gch]], rows_i.at[dst], sem))
            copies.append(pltpu.async_copy(
                sj_h.at[pair_j.at[gch]], rows_j.at[dst], sem))
        for cp in copies:
            cp.wait()

        for g in range(ppb // _LANES):
            gch = p * cpp + g * _LANES // _CHUNK
            isl = pl.ds((g * _LANES) % _CHUNK, _LANES)
            rows16 = g * _LANES + lanes
            half_u = (inv_u[gch, isl] & 1) * _D
            half_i = (inv_i[gch, isl] & 1) * _D
            half_j = (inv_j[gch, isl] & 1) * _D
            zero = jnp.zeros((_LANES,), jnp.float32)

            def d_body(d, carry):
                acc_i, acc_j, cu, ci, cj = carry
                u = plsc.load_gather(rows_u, [rows16, cu])
                vi = plsc.load_gather(rows_i, [rows16, ci])
                vj = plsc.load_gather(rows_j, [rows16, cj])
                return (acc_i + u * vi, acc_j + u * vj,
                        cu + 1, ci + 1, cj + 1)

            acc_i, acc_j, _, _, _ = lax.fori_loop(
                0, _D, d_body, (zero, zero, half_u, half_i, half_j),
                unroll=8)
            out = pl.ds(p * ppb + g * _LANES, _LANES)
            out_i_v[out] = acc_i
            out_j_v[out] = acc_j

    pltpu.sync_copy(out_i_v, out_i_h.at[pl.ds(base, bpw)])
    pltpu.sync_copy(out_j_v, out_j_h.at[pl.ds(base, bpw)])


_dot_call = pl.kernel(
    _dot_body,
    out_type=(
        jax.ShapeDtypeStruct((_B,), jnp.float32),
        jax.ShapeDtypeStruct((_B,), jnp.float32),
    ),
    mesh=plsc.VectorSubcoreMesh(
        core_axis_name="c", subcore_axis_name="s",
        num_cores=_NC, num_subcores=_NS,
    ),
    compiler_params=pltpu.CompilerParams(
        needs_layout_passes=False, use_tc_tiling_on_sc=True),
    scratch_types=[
        pltpu.VMEM((4, _CHUNK), jnp.int32),
        pltpu.VMEM((4, _CHUNK), jnp.int32),
        pltpu.VMEM((4, _CHUNK), jnp.int32),
        pltpu.VMEM((4, _CHUNK), jnp.int32),
        pltpu.VMEM((4, _CHUNK), jnp.int32),
        pltpu.VMEM((4, _CHUNK), jnp.int32),
        pltpu.VMEM((256, _W), jnp.float32),
        pltpu.VMEM((256, _W), jnp.float32),
        pltpu.VMEM((256, _W), jnp.float32),
        pltpu.VMEM((_B // _NW,), jnp.float32),
        pltpu.VMEM((_B // _NW,), jnp.float32),
        pltpu.SemaphoreType.DMA,
    ],
)


def _routing(sorted_idx, n_per_tile):
    n = sorted_idx.shape[0]
    pan = (sorted_idx >> 7).astype(jnp.int32)
    pos = jnp.arange(n, dtype=jnp.int32)
    prev = jnp.concatenate([jnp.full((1,), -1, jnp.int32), pan[:-1]])
    new = (pan != prev) | (pos % n_per_tile == 0)
    slot_global = jnp.cumsum(new.astype(jnp.int32)) - 1
    tile = pos // n_per_tile
    seg_start = slot_global[tile * n_per_tile]
    slot = (slot_global - seg_start).astype(jnp.int32)
    ntiles = n // n_per_tile
    fetch = jnp.zeros((ntiles, n_per_tile), jnp.int32)
    fetch = fetch.at[tile, slot].set(pan)
    nf = jnp.zeros((ntiles,), jnp.int32).at[tile].max(slot + 1)
    nf2 = jnp.broadcast_to(nf[:, None], (ntiles, _CHUNK))
    rl = (sorted_idx & 127).astype(jnp.int32)
    c2 = (ntiles * n_per_tile) // _CHUNK
    return (rl.reshape(c2, _CHUNK), slot.reshape(c2, _CHUNK),
            fetch.reshape(c2, _CHUNK), nf2)


def kernel(user, item_i, item_j, embed_user_weight, embed_item_weight):
    user = user.astype(jnp.int32)
    item_i = item_i.astype(jnp.int32)
    item_j = item_j.astype(jnp.int32)
    uwt = embed_user_weight.T
    iwt = embed_item_weight.T

    iota_b = jnp.arange(_B, dtype=jnp.int32)
    iota_2b = jnp.arange(2 * _B, dtype=jnp.int32)
    su, pu = lax.sort_key_val(user, iota_b)
    cat = jnp.concatenate([item_i, item_j])
    sc_, pc = lax.sort_key_val(cat, iota_2b)
    inv_u = jnp.zeros((_B,), jnp.int32).at[pu].set(iota_b)
    inv_cat = jnp.zeros((2 * _B,), jnp.int32).at[pc].set(iota_2b)
    inv_i, inv_j = inv_cat[:_B], inv_cat[_B:]

    ru = _routing(su, _UPT)
    rc = _routing(sc_, _CPT)

    stag_u = _stream_user(uwt, *ru)
    stag_c = _stream_item(iwt, *rc)

    sh = (_B // _CHUNK, _CHUNK)
    return _dot_call(inv_u.reshape(sh), inv_i.reshape(sh),
                     inv_j.reshape(sh), stag_u, stag_c, stag_c)


# speedup vs baseline: 3.7234x; 1.7405x over previous
import jax
import jax.numpy as jnp
from jax import lax
from jax.experimental import pallas as pl
from jax.experimental.pallas import tpu as pltpu
from jax.experimental.pallas import tpu_sc as plsc

_B = 16384
_D = 64
_W = 128
_NC = 2
_NS = 16
_NW = _NC * _NS
_LANES = 16
_NBUF = 4

_UPT = _B // _NW
_CPT = 2 * _B // _NW
_CHUNK = 128


def _wid():
    return lax.axis_index("s") * _NC + lax.axis_index("c")


def _stream_body(table_h, rl_v_h, slot_v_h, fetch_h, nf_h, stag_h,
                 rl_v, slot_v, fetch_v, nf_v, rows_out, ring, sem, osem,
                 *, n_per_tile):
    nchunk = n_per_tile // _CHUNK
    w = _wid()
    pltpu.sync_copy(rl_v_h.at[pl.ds(w * nchunk, nchunk)], rl_v)
    pltpu.sync_copy(slot_v_h.at[pl.ds(w * nchunk, nchunk)], slot_v)
    pltpu.sync_copy(fetch_h.at[pl.ds(w * nchunk, nchunk)], fetch_v)
    pltpu.sync_copy(nf_h.at[pl.ds(w, 1)], nf_v)

    lanes = lax.iota(jnp.int32, _LANES)

    def bcast(x):
        return jnp.full((_LANES,), x, jnp.int32)

    def fire(s):
        c = plsc.load_gather(
            fetch_v, [bcast(s >> 7), bcast(s & (_CHUNK - 1))])[0]
        pltpu.async_copy(
            table_h.at[:, pl.ds(c * _CHUNK, _CHUNK)],
            ring.at[s % _NBUF], sem)

    nf = nf_v[0, pl.ds(0, _LANES)][0]
    for s0 in range(_NBUF - 1):
        @pl.when(s0 < nf)
        def _():
            fire(jnp.int32(s0))

    def group_body(g, cur):
        ch = g >> 3
        off = (g & 7) * _LANES + lanes
        slot16 = plsc.load_gather(slot_v, [bcast(ch), off])
        rl16 = plsc.load_gather(rl_v, [bcast(ch), off])
        for k in range(_LANES):
            slot = slot16[k]

            def advance(c):
                pltpu.make_async_copy(
                    table_h.at[:, pl.ds(0, _CHUNK)], ring.at[0], sem
                ).wait()
                @pl.when(c + _NBUF - 1 < nf)
                def _():
                    fire(c + _NBUF - 1)
                return c + 1

            cur = lax.while_loop(lambda c: c <= slot, advance, cur)
            ringsel = bcast(slot % _NBUF)
            col = bcast(rl16[k])
            row16 = bcast(g * 8 + (k >> 1))
            for qq in range(_D // _LANES):
                vals = plsc.load_gather(
                    ring, [ringsel, qq * _LANES + lanes, col])
                plsc.store_scatter(
                    rows_out,
                    [row16, (k & 1) * _D + qq * _LANES + lanes], vals)
        return cur

    lax.fori_loop(0, n_per_tile // _LANES, group_body, jnp.int32(0))

    out_rows = n_per_tile // 2
    pltpu.async_copy(
        rows_out, stag_h.at[pl.ds(w * out_rows, out_rows)], osem).wait()


def _make_stream(n_per_tile, n_total):
    nchunk = n_per_tile // _CHUNK
    import functools
    body = functools.partial(_stream_body, n_per_tile=n_per_tile)
    return pl.kernel(
        body,
        out_type=jax.ShapeDtypeStruct((n_total // 2, _W), jnp.float32),
        mesh=plsc.VectorSubcoreMesh(
            core_axis_name="c", subcore_axis_name="s",
            num_cores=_NC, num_subcores=_NS,
        ),
        compiler_params=pltpu.CompilerParams(
            needs_layout_passes=False, use_tc_tiling_on_sc=True),
        scratch_types=[
            pltpu.VMEM((nchunk, _CHUNK), jnp.int32),
            pltpu.VMEM((nchunk, _CHUNK), jnp.int32),
            pltpu.VMEM((nchunk, _CHUNK), jnp.int32),
            pltpu.VMEM((1, _CHUNK), jnp.int32),
            pltpu.VMEM((n_per_tile // 2, _W), jnp.float32),
            pltpu.VMEM((_NBUF, _D, _CHUNK), jnp.float32),
            pltpu.SemaphoreType.DMA,
            pltpu.SemaphoreType.DMA,
        ],
    )


_stream_user = _make_stream(_UPT, _B)
_stream_item = _make_stream(_CPT, 2 * _B)


def _dot_body(inv_u_h, inv_i_h, inv_j_h, su_h, si_h, sj_h, out_i_h, out_j_h,
              inv_u, inv_i, inv_j, pair_u, pair_i, pair_j,
              rows_u, rows_i, rows_j, out_i_v, out_j_v, sem):
    w = _wid()
    bpw = _B // _NW
    nchunk = bpw // _CHUNK
    base = w * bpw

    pltpu.sync_copy(inv_u_h.at[pl.ds(w * nchunk, nchunk)], inv_u)
    pltpu.sync_copy(inv_i_h.at[pl.ds(w * nchunk, nchunk)], inv_i)
    pltpu.sync_copy(inv_j_h.at[pl.ds(w * nchunk, nchunk)], inv_j)

    for ch in range(nchunk):
        for v in range(_CHUNK // _LANES):
            sl = pl.ds(v * _LANES, _LANES)
            pair_u[ch, sl] = inv_u[ch, sl] >> 1
            pair_i[ch, sl] = inv_i[ch, sl] >> 1
            pair_j[ch, sl] = inv_j[ch, sl] >> 1

    lanes = lax.iota(jnp.int32, _LANES)
    nphase = 2
    ppb = bpw // nphase
    cpp = ppb // _CHUNK

    for p in range(nphase):
        copies = []
        for ch in range(cpp):
            gch = p * cpp + ch
            dst = pl.ds(ch * _CHUNK, _CHUNK)
            copies.append(pltpu.async_copy(
                su_h.at[pair_u.at[gch]], rows_u.at[dst], sem))
            copies.append(pltpu.async_copy(
                si_h.at[pair_i.at[gch]], rows_i.at[dst], sem))
            copies.append(pltpu.async_copy(
                sj_h.at[pair_j.at[gch]], rows_j.at[dst], sem))
        for cp in copies:
            cp.wait()

        for g in range(ppb // _LANES):
            gch = p * cpp + g * _LANES // _CHUNK
            isl = pl.ds((g * _LANES) % _CHUNK, _LANES)
            rows16 = g * _LANES + lanes
            half_u = (inv_u[gch, isl] & 1) * _D
            half_i = (inv_i[gch, isl] & 1) * _D
            half_j = (inv_j[gch, isl] & 1) * _D
            zero = jnp.zeros((_LANES,), jnp.float32)

            def d_body(d, carry):
                acc_i, acc_j, cu, ci, cj = carry
                u = plsc.load_gather(rows_u, [rows16, cu])
                vi = plsc.load_gather(rows_i, [rows16, ci])
                vj = plsc.load_gather(rows_j, [rows16, cj])
                return (acc_i + u * vi, acc_j + u * vj,
                        cu + 1, ci + 1, cj + 1)

            acc_i, acc_j, _, _, _ = lax.fori_loop(
                0, _D, d_body, (zero, zero, half_u, half_i, half_j),
                unroll=8)
            out = pl.ds(p * ppb + g * _LANES, _LANES)
            out_i_v[out] = acc_i
            out_j_v[out] = acc_j

    pltpu.sync_copy(out_i_v, out_i_h.at[pl.ds(base, bpw)])
    pltpu.sync_copy(out_j_v, out_j_h.at[pl.ds(base, bpw)])


_dot_call = pl.kernel(
    _dot_body,
    out_type=(
        jax.ShapeDtypeStruct((_B,), jnp.float32),
        jax.ShapeDtypeStruct((_B,), jnp.float32),
    ),
    mesh=plsc.VectorSubcoreMesh(
        core_axis_name="c", subcore_axis_name="s",
        num_cores=_NC, num_subcores=_NS,
    ),
    compiler_params=pltpu.CompilerParams(
        needs_layout_passes=False, use_tc_tiling_on_sc=True),
    scratch_types=[
        pltpu.VMEM((4, _CHUNK), jnp.int32),
        pltpu.VMEM((4, _CHUNK), jnp.int32),
        pltpu.VMEM((4, _CHUNK), jnp.int32),
        pltpu.VMEM((4, _CHUNK), jnp.int32),
        pltpu.VMEM((4, _CHUNK), jnp.int32),
        pltpu.VMEM((4, _CHUNK), jnp.int32),
        pltpu.VMEM((256, _W), jnp.float32),
        pltpu.VMEM((256, _W), jnp.float32),
        pltpu.VMEM((256, _W), jnp.float32),
        pltpu.VMEM((_B // _NW,), jnp.float32),
        pltpu.VMEM((_B // _NW,), jnp.float32),
        pltpu.SemaphoreType.DMA,
    ],
)


def _routing(sorted_idx, n_per_tile):
    n = sorted_idx.shape[0]
    ntiles = n // n_per_tile
    pan = (sorted_idx >> 7).astype(jnp.int32)
    pos = jnp.arange(n, dtype=jnp.int32)
    prev = jnp.concatenate([jnp.full((1,), -1, jnp.int32), pan[:-1]])
    new = (pan != prev) | (pos % n_per_tile == 0)
    slot_global = jnp.cumsum(new.astype(jnp.int32)) - 1
    sg2 = slot_global.reshape(ntiles, n_per_tile)
    slot = (sg2 - sg2[:, :1]).astype(jnp.int32).reshape(n)
    nf = (sg2[:, -1:] - sg2[:, :1] + 1).astype(jnp.int32)
    nf2 = jnp.broadcast_to(nf, (ntiles, _CHUNK))
    sent = jnp.int32(16383)
    tile = pos // n_per_tile
    key = tile * jnp.int32(16384) + jnp.where(new, pan, sent)
    fetch = lax.sort(key) & sent
    rl = (sorted_idx & 127).astype(jnp.int32)
    c2 = n // _CHUNK
    return (rl.reshape(c2, _CHUNK), slot.reshape(c2, _CHUNK),
            fetch.reshape(c2, _CHUNK), nf2)


def kernel(user, item_i, item_j, embed_user_weight, embed_item_weight):
    user = user.astype(jnp.int32)
    item_i = item_i.astype(jnp.int32)
    item_j = item_j.astype(jnp.int32)
    uwt = embed_user_weight.T
    iwt = embed_item_weight.T

    iota_b = jnp.arange(_B, dtype=jnp.int32)
    iota_2b = jnp.arange(2 * _B, dtype=jnp.int32)
    su, pu = lax.sort_key_val(user, iota_b)
    cat = jnp.concatenate([item_i, item_j])
    sc_, pc = lax.sort_key_val(cat, iota_2b)
    _, inv_u = lax.sort_key_val(pu, iota_b)
    _, inv_cat = lax.sort_key_val(pc, iota_2b)
    inv_i, inv_j = inv_cat[:_B], inv_cat[_B:]

    ru = _routing(su, _UPT)
    rc = _routing(sc_, _CPT)

    stag_u = _stream_user(uwt, *ru)
    stag_c = _stream_item(iwt, *rc)

    sh = (_B // _CHUNK, _CHUNK)
    return _dot_call(inv_u.reshape(sh), inv_i.reshape(sh),
                     inv_j.reshape(sh), stag_u, stag_c, stag_c)
